# Initial kernel scaffold; baseline (speedup 1.0000x reference)
import jax
import jax.numpy as jnp
from jax.experimental import pallas as pl


def kernel(input, weight):
    raise NotImplementedError("write your pallas kernel here")


# speedup vs baseline: 1.0933x; 1.0933x over previous
import functools

import jax
import jax.numpy as jnp
from jax import lax
from jax.experimental import pallas as pl
from jax.experimental.pallas import tpu as pltpu
from jax.experimental.pallas import tpu_sc as plsc

NUM_EMB = 1_000_000
DIM = 32
BATCH = 16384
HIST = 50
B_TOTAL = BATCH * HIST

NC = 2
NS = 16
NW = NC * NS
BPW = B_TOTAL // NW

SUB = 128
K = 8
CHUNK = SUB * K
NCHUNK = BPW // CHUNK

_mesh = plsc.VectorSubcoreMesh(core_axis_name="c", subcore_axis_name="s")


@functools.partial(
    pl.kernel,
    out_type=jax.ShapeDtypeStruct((B_TOTAL, DIM), jnp.float32),
    mesh=_mesh,
    scratch_types=[
        pltpu.VMEM((K, SUB), jnp.int32),
        pltpu.VMEM((CHUNK, DIM), jnp.float32),
        pltpu.SemaphoreType.DMA,
    ],
    compiler_params=pltpu.CompilerParams(use_tc_tiling_on_sc=False),
)
def _gather_kernel(table_hbm, idx_hbm, out_hbm, idx_v, rows_v, sem):
    wid = lax.axis_index("s") * NC + lax.axis_index("c")
    base = wid * BPW

    def body(i, carry):
        off = base + i * CHUNK
        row = pl.multiple_of(off // SUB, 8)
        pltpu.sync_copy(idx_hbm.at[pl.ds(row, K), :], idx_v)
        for j in range(K):
            pltpu.async_copy(
                table_hbm.at[idx_v.at[j]],
                rows_v.at[pl.ds(j * SUB, SUB)],
                sem,
            )
        for j in range(K):
            pltpu.make_async_copy(
                table_hbm.at[idx_v.at[j]],
                rows_v.at[pl.ds(j * SUB, SUB)],
                sem,
            ).wait()
        pltpu.sync_copy(rows_v, out_hbm.at[pl.ds(off, CHUNK)])
        return carry

    lax.fori_loop(0, NCHUNK, body, 0)


def kernel(input, weight):
    idx2d = input.reshape(B_TOTAL // SUB, SUB).astype(jnp.int32)
    out = _gather_kernel(weight, idx2d)
    return out.reshape(BATCH, HIST, DIM)


# speedup vs baseline: 1.1092x; 1.0145x over previous
import functools

import jax
import jax.numpy as jnp
from jax import lax
from jax.experimental import pallas as pl
from jax.experimental.pallas import tpu as pltpu
from jax.experimental.pallas import tpu_sc as plsc

NUM_EMB = 1_000_000
DIM = 32
BATCH = 16384
HIST = 50
B_TOTAL = BATCH * HIST

NC = 2
NS = 16
NW = NC * NS
BPW = B_TOTAL // NW

SUB = 128
K = 8
CHUNK = SUB * K
NCHUNK = BPW // CHUNK
IDX_ROWS = B_TOTAL // SUB
RPW = IDX_ROWS // NW

_mesh = plsc.VectorSubcoreMesh(core_axis_name="c", subcore_axis_name="s")


@functools.partial(
    pl.kernel,
    out_type=jax.ShapeDtypeStruct((B_TOTAL, DIM), jnp.float32),
    mesh=_mesh,
    scratch_types=[
        pltpu.VMEM((K, SUB), jnp.int32),
        pltpu.VMEM((K, SUB), jnp.int32),
        pltpu.VMEM((CHUNK, DIM), jnp.float32),
        pltpu.VMEM((CHUNK, DIM), jnp.float32),
        pltpu.SemaphoreType.DMA,
        pltpu.SemaphoreType.DMA,
        pltpu.SemaphoreType.DMA,
        pltpu.SemaphoreType.DMA,
        pltpu.SemaphoreType.DMA,
        pltpu.SemaphoreType.DMA,
    ],
    compiler_params=pltpu.CompilerParams(use_tc_tiling_on_sc=False),
)
def _gather_kernel(table_hbm, idx_hbm, out_hbm,
                   idx_v0, idx_v1, rows_v0, rows_v1,
                   s_idx0, s_idx1, s_gat0, s_gat1, s_out0, s_out1):
    idx_v = (idx_v0, idx_v1)
    rows_v = (rows_v0, rows_v1)
    s_idx = (s_idx0, s_idx1)
    s_gat = (s_gat0, s_gat1)
    s_out = (s_out0, s_out1)

    wid = lax.axis_index("s") * NC + lax.axis_index("c")
    base = wid * BPW
    base_row = wid * RPW

    def fire_idx(i, slot):
        row = pl.multiple_of(base_row + i * K, 8)
        pltpu.async_copy(idx_hbm.at[pl.ds(row, K), :], idx_v[slot], s_idx[slot])

    def wait_idx(slot):
        pltpu.make_async_copy(
            idx_hbm.at[pl.ds(pl.multiple_of(base_row, 8), K), :],
            idx_v[slot], s_idx[slot]).wait()

    def fire_gathers(slot):
        for j in range(K):
            pltpu.async_copy(
                table_hbm.at[idx_v[slot].at[j]],
                rows_v[slot].at[pl.ds(j * SUB, SUB)],
                s_gat[slot])

    def wait_gathers(slot):
        for j in range(K):
            pltpu.make_async_copy(
                table_hbm.at[idx_v[slot].at[j]],
                rows_v[slot].at[pl.ds(j * SUB, SUB)],
                s_gat[slot]).wait()

    def fire_out(i, slot):
        off = base + i * CHUNK
        pltpu.async_copy(rows_v[slot], out_hbm.at[pl.ds(off, CHUNK)], s_out[slot])

    def wait_out(slot):
        pltpu.make_async_copy(
            rows_v[slot], out_hbm.at[pl.ds(base, CHUNK)], s_out[slot]).wait()

    def visit(i, j):
        k = 1 - j
        wait_gathers(k)
        fire_idx(i + 1, k)
        wait_out(j)
        wait_idx(j)
        fire_gathers(j)
        fire_out(i - 1, k)

    fire_idx(0, 0)
    wait_idx(0)
    fire_gathers(0)
    fire_idx(1, 1)

    wait_gathers(0)
    fire_idx(2, 0)
    wait_idx(1)
    fire_gathers(1)
    fire_out(0, 0)

    wait_gathers(1)
    fire_idx(3, 1)
    wait_out(0)
    wait_idx(0)
    fire_gathers(0)
    fire_out(1, 1)

    visit(3, 1)

    def body(t, carry):
        i = 4 + 2 * t
        visit(i, 0)
        visit(i + 1, 1)
        return carry

    lax.fori_loop(0, (NCHUNK - 5) // 2, body, 0)

    wait_gathers(1)
    wait_out(0)
    wait_idx(0)
    fire_gathers(0)
    fire_out(NCHUNK - 2, 1)

    wait_gathers(0)
    fire_out(NCHUNK - 1, 0)
    wait_out(1)
    wait_out(0)


def kernel(input, weight):
    idx2d = input.reshape(IDX_ROWS, SUB).astype(jnp.int32)
    out = _gather_kernel(weight, idx2d)
    return out.reshape(BATCH, HIST, DIM)


# speedup vs baseline: 1.2554x; 1.1318x over previous
import functools

import jax
import jax.numpy as jnp
from jax import lax
from jax.experimental import pallas as pl
from jax.experimental.pallas import tpu as pltpu
from jax.experimental.pallas import tpu_sc as plsc

NUM_EMB = 1_000_000
DIM = 32
BATCH = 16384
HIST = 50
HP = 56
ESUP = NUM_EMB // 4
SUPW = 128

NC = 2
NS = 16
NW = NC * NS
CPW = (BATCH // 128) // NW
NSUB = CPW * HIST

_mesh = plsc.VectorSubcoreMesh(core_axis_name="c", subcore_axis_name="s")


@functools.partial(
    pl.kernel,
    out_type=jax.ShapeDtypeStruct((HIST, DIM, BATCH), jnp.float32),
    mesh=_mesh,
    scratch_types=[
        pltpu.VMEM((8, 128), jnp.int32),
        pltpu.VMEM((128,), jnp.int32),
        pltpu.VMEM((128, SUPW), jnp.float32),
        pltpu.VMEM((DIM, 128), jnp.float32),
        pltpu.SemaphoreType.DMA,
    ],
    compiler_params=pltpu.CompilerParams(needs_layout_passes=False),
)
def _gather_kernel(wsup_hbm, idx_hbm, out_hbm,
                   idx_t, sup_idx, sup_rows, trans, s_gat):
    wid = lax.axis_index("s") * NC + lax.axis_index("c")
    iota16 = lax.iota(jnp.int32, 16)

    def body(s, carry):
        c = s // HIST
        r = s % HIST
        j = r % 8
        b0 = (wid * CPW + c) * 128

        @pl.when(j == 0)
        def _fetch_tile():
            pltpu.sync_copy(
                idx_hbm.at[pl.ds(pl.multiple_of(r, 8), 8), pl.ds(b0, 128)],
                idx_t)

        for j2 in range(8):
            v = idx_t[j, pl.ds(j2 * 16, 16)]
            sup_idx[pl.ds(j2 * 16, 16)] = v >> 2

        pltpu.async_copy(wsup_hbm.at[sup_idx], sup_rows, s_gat)
        pltpu.make_async_copy(wsup_hbm.at[sup_idx], sup_rows, s_gat).wait()

        for j2 in range(8):
            v = idx_t[j, pl.ds(j2 * 16, 16)]
            ext = (v & 3) * 32
            row = j2 * 16 + iota16

            def dbody(d, carry2):
                g = plsc.load_gather(sup_rows, [row, ext + d])
                trans[d, pl.ds(j2 * 16, 16)] = g
                return carry2

            lax.fori_loop(0, DIM, dbody, 0)

        pltpu.sync_copy(trans, out_hbm.at[r, :, pl.ds(b0, 128)])
        return carry

    lax.fori_loop(0, NSUB, body, 0)


def kernel(input, weight):
    idxp = jnp.pad(input.T, ((0, HP - HIST), (0, 0)))
    wsup = weight.reshape(ESUP, SUPW)
    outk = _gather_kernel(wsup, idxp)
    return outk.transpose(2, 0, 1)


# speedup vs baseline: 1.6201x; 1.2905x over previous
import functools

import jax
import jax.numpy as jnp
from jax import lax
from jax.experimental import pallas as pl
from jax.experimental.pallas import tpu as pltpu
from jax.experimental.pallas import tpu_sc as plsc

NUM_EMB = 1_000_000
DIM = 32
BATCH = 16384
HIST = 50
HP = 56
ESUP = NUM_EMB // 4
SUPW = 128

NC = 2
NS = 16
NW = NC * NS
CPW = (BATCH // 128) // NW
NSUB = CPW * HIST

_mesh = plsc.VectorSubcoreMesh(core_axis_name="c", subcore_axis_name="s")


@functools.partial(
    pl.kernel,
    out_type=jax.ShapeDtypeStruct((HIST, DIM, BATCH), jnp.float32),
    mesh=_mesh,
    scratch_types=[
        pltpu.VMEM((8, 128), jnp.int32),
        pltpu.VMEM((128,), jnp.int32),
        pltpu.VMEM((128,), jnp.int32),
        pltpu.VMEM((128,), jnp.int32),
        pltpu.VMEM((128,), jnp.int32),
        pltpu.VMEM((128, SUPW), jnp.float32),
        pltpu.VMEM((128, SUPW), jnp.float32),
        pltpu.VMEM((DIM, 128), jnp.float32),
        pltpu.VMEM((DIM, 128), jnp.float32),
        pltpu.SemaphoreType.DMA,
        pltpu.SemaphoreType.DMA,
        pltpu.SemaphoreType.DMA,
        pltpu.SemaphoreType.DMA,
        pltpu.SemaphoreType.DMA,
    ],
    compiler_params=pltpu.CompilerParams(needs_layout_passes=False),
)
def _gather_kernel(wsup_hbm, idx_hbm, out_hbm,
                   idx_t, sup_idx0, sup_idx1, ext_b0, ext_b1,
                   sup_rows0, sup_rows1, trans0, trans1,
                   s_idx, s_gat0, s_gat1, s_out0, s_out1):
    sup_idx = (sup_idx0, sup_idx1)
    ext_b = (ext_b0, ext_b1)
    sup_rows = (sup_rows0, sup_rows1)
    trans = (trans0, trans1)
    s_gat = (s_gat0, s_gat1)
    s_out = (s_out0, s_out1)

    wid = lax.axis_index("s") * NC + lax.axis_index("c")
    iota16 = lax.iota(jnp.int32, 16)

    def idx_fetch_descr(s):
        c = s // HIST
        r = s % HIST
        b0 = (wid * CPW + c) * 128
        return (idx_hbm.at[pl.ds(pl.multiple_of(r - r % 8, 8), 8),
                           pl.ds(b0, 128)], idx_t, s_idx)

    def stage(s, p):
        r = s % HIST
        j = r % 8

        @pl.when(j == 0)
        def _wait_tile():
            pltpu.make_async_copy(*idx_fetch_descr(s)).wait()

        for j2 in range(8):
            v = idx_t[j, pl.ds(j2 * 16, 16)]
            sup_idx[p][pl.ds(j2 * 16, 16)] = v >> 2
            ext_b[p][pl.ds(j2 * 16, 16)] = (v & 3) * 32
        pltpu.async_copy(wsup_hbm.at[sup_idx[p]], sup_rows[p], s_gat[p])

        @pl.when(jnp.logical_and((s + 1) % HIST % 8 == 0, s < NSUB - 1))
        def _prefetch_tile():
            pltpu.async_copy(*idx_fetch_descr(s + 1))

    def drain(s, p, wait_write):
        c = s // HIST
        r = s % HIST
        b0 = (wid * CPW + c) * 128
        pltpu.make_async_copy(wsup_hbm.at[sup_idx[p]], sup_rows[p],
                              s_gat[p]).wait()
        if wait_write:
            pltpu.make_async_copy(
                trans[p], out_hbm.at[0, :, pl.ds(0, 128)], s_out[p]).wait()
        for j2 in range(8):
            ext = ext_b[p][pl.ds(j2 * 16, 16)]
            row = j2 * 16 + iota16

            def dbody(d, carry2):
                g = plsc.load_gather(sup_rows[p], [row, ext + d])
                trans[p][d, pl.ds(j2 * 16, 16)] = g
                return carry2

            lax.fori_loop(0, DIM, dbody, 0)
        pltpu.async_copy(trans[p], out_hbm.at[r, :, pl.ds(b0, 128)], s_out[p])

    pltpu.async_copy(*idx_fetch_descr(0))
    stage(0, 0)
    stage(1, 1)
    drain(0, 0, False)
    stage(2, 0)
    drain(1, 1, False)

    def body(t, carry):
        s = 3 + 2 * t
        stage(s, 1)
        drain(s - 1, 0, True)
        stage(s + 1, 0)
        drain(s, 1, True)
        return carry

    lax.fori_loop(0, (NSUB - 4) // 2, body, 0)

    stage(NSUB - 1, 1)
    drain(NSUB - 2, 0, True)
    drain(NSUB - 1, 1, True)
    pltpu.make_async_copy(trans0, out_hbm.at[0, :, pl.ds(0, 128)], s_out0).wait()
    pltpu.make_async_copy(trans1, out_hbm.at[0, :, pl.ds(0, 128)], s_out1).wait()


def kernel(input, weight):
    idxp = jnp.pad(input.T, ((0, HP - HIST), (0, 0)))
    wsup = weight.reshape(ESUP, SUPW)
    outk = _gather_kernel(wsup, idxp)
    return outk.transpose(2, 0, 1)
